# baseline (speedup 1.0000x reference)
import functools

import numpy as np
import jax
import jax.numpy as jnp
from jax import lax
from jax.experimental import pallas as pl
from jax.experimental.pallas import tpu as pltpu
from jax.experimental.pallas import tpu_sc as plsc

B = 32
C = 3
H = 384
WI = 384
P = 16
NS = 24
NP_ = NS * NS
D = 768
IN_FEAT = P * P * C
LEN_KEEP = 144
RPP = C * P
RPB = LEN_KEEP * RPP
IDX_MINOR = 128
N_CHUNK = RPB // IDX_MINOR
HALF_CHUNK = N_CHUNK // 2
HALF_ROWS = HALF_CHUNK * IDX_MINOR
POS_CHUNK = 72


def _patch_hw_rows():
    j = lax.broadcasted_iota(jnp.int32, (1, NP_), 1)
    return (j // NS).astype(jnp.float32), (j % NS).astype(jnp.float32)


def _snap_cols(b4):
    s = jnp.round(b4 * float(NS)) / float(NS)
    x0 = jnp.clip(s[:, 0:1], 0.0, 1.0 - 1.0 / NS)
    y0 = jnp.clip(s[:, 1:2], 0.0, 1.0 - 1.0 / NS)
    x1 = jnp.clip(jnp.maximum(s[:, 2:3], x0 + 1.0 / NS), 0.0, 1.0)
    y1 = jnp.clip(jnp.maximum(s[:, 3:4], y0 + 1.0 / NS), 0.0, 1.0)
    return x0, y0, x1, y1


def _a_body(n3_ref, bb3_ref, fb3_ref, restore_ref, mask_ref, keep_ref,
            pos_ref, rix_ref, face_ref):
    b = pl.program_id(0)
    bbr = bb3_ref[0]
    fbr = fb3_ref[0]
    x0, y0, x1, y1 = _snap_cols(bbr)
    bw = x1 - x0
    bh = y1 - y0
    f0 = (fbr[:, 0:1] - x0) / bw
    f1 = (fbr[:, 1:2] - y0) / bh
    f2 = (fbr[:, 2:3] - x0) / bw
    f3 = (fbr[:, 3:4] - y0) / bh
    fcl = jnp.clip(jnp.concatenate([f0, f1, f2, f3], axis=1), 0.0, 1.0)
    g0, g1, g2, g3 = _snap_cols(fcl)
    face_ref[...] = jnp.concatenate(
        [g0 * bw + x0, g1 * bh + y0, g2 * bw + x0, g3 * bh + y0], axis=1)[None]
    yb, xb = _patch_hw_rows()
    gx = (xb + 0.5) / NS
    gy = (yb + 0.5) / NS
    in_x = (gx >= x0) & (gx < x1)
    in_y = (gy >= y0) & (gy < y1)
    mx = jnp.clip(jnp.floor((gx - x0) / bw * NS), 0.0, float(NS - 1))
    my = jnp.clip(jnp.floor((gy - y0) / bh * NS), 0.0, float(NS - 1))
    ix = jnp.where(in_x, mx, xb)
    iy = jnp.where(in_y, my, yb)
    flat_row = iy * float(NS) + ix
    nrow = n3_ref[0]
    ncol = jnp.swapaxes(nrow, 0, 1)
    ilt = (lax.broadcasted_iota(jnp.int32, (NP_, NP_), 0)
           < lax.broadcasted_iota(jnp.int32, (NP_, NP_), 1))
    lt = ncol < nrow
    eq = ncol == nrow
    cmpf = (lt | (eq & ilt)).astype(jnp.float32)
    rank_row = jnp.sum(cmpf, axis=0, keepdims=True)
    restore_ref[...] = rank_row.astype(jnp.int32)[None]
    mask_ref[...] = (rank_row >= float(LEN_KEEP)).astype(jnp.float32)[None]
    r_row = lax.broadcasted_iota(jnp.int32, (1, LEN_KEEP), 1
                                 ).astype(jnp.float32)
    rank_col = jnp.swapaxes(rank_row, 0, 1)
    flat_col = jnp.swapaxes(flat_row, 0, 1)
    m3 = (rank_col == r_row).astype(jnp.float32)
    jc = lax.broadcasted_iota(jnp.int32, (NP_, 1), 0)
    j_col = jc.astype(jnp.float32)
    h_col = (jc // NS).astype(jnp.float32)
    w_col = (jc % NS).astype(jnp.float32)
    keep_row = jnp.sum(m3 * j_col, axis=0, keepdims=True)
    pos_row = jnp.sum(m3 * flat_col, axis=0, keepdims=True)
    h_row2 = jnp.sum(m3 * h_col, axis=0, keepdims=True)
    w_row2 = jnp.sum(m3 * w_col, axis=0, keepdims=True)
    keep_ref[...] = keep_row.astype(jnp.int32)[None]
    pos_ref[...] = pos_row.astype(jnp.int32)[None]
    cp = lax.broadcasted_iota(jnp.int32, (1, RPP), 1)
    base48 = ((cp // P) * (H * NS) + (cp % P) * NS).astype(jnp.float32)
    hw_col = jnp.swapaxes(h_row2 * float(P * NS) + w_row2, 0, 1)
    rix = (hw_col + base48).astype(jnp.int32) + b * (C * H * NS)
    rix_ref[...] = rix[None]


_a = pl.pallas_call(
    _a_body,
    grid=(B,),
    in_specs=[pl.BlockSpec((1, 1, NP_), lambda b: (b, 0, 0)),
              pl.BlockSpec((1, 1, 4), lambda b: (b, 0, 0)),
              pl.BlockSpec((1, 1, 4), lambda b: (b, 0, 0))],
    out_specs=[pl.BlockSpec((1, 1, NP_), lambda b: (b, 0, 0)),
               pl.BlockSpec((1, 1, NP_), lambda b: (b, 0, 0)),
               pl.BlockSpec((1, 1, LEN_KEEP), lambda b: (b, 0, 0)),
               pl.BlockSpec((1, 1, LEN_KEEP), lambda b: (b, 0, 0)),
               pl.BlockSpec((1, LEN_KEEP, RPP), lambda b: (b, 0, 0)),
               pl.BlockSpec((1, 1, 4), lambda b: (b, 0, 0))],
    out_shape=(jax.ShapeDtypeStruct((B, 1, NP_), jnp.int32),
               jax.ShapeDtypeStruct((B, 1, NP_), jnp.float32),
               jax.ShapeDtypeStruct((B, 1, LEN_KEEP), jnp.int32),
               jax.ShapeDtypeStruct((B, 1, LEN_KEEP), jnp.int32),
               jax.ShapeDtypeStruct((B, LEN_KEEP, RPP), jnp.int32),
               jax.ShapeDtypeStruct((B, 1, 4), jnp.float32)),
)


_NC = 2


@functools.lru_cache(maxsize=None)
def _sc_gather():
    mesh = plsc.VectorSubcoreMesh(core_axis_name="c", subcore_axis_name="s")

    @functools.partial(
        pl.kernel,
        out_type=(jax.ShapeDtypeStruct((B, 2, HALF_ROWS, P), jnp.float32),
                  jax.ShapeDtypeStruct((B, 2, POS_CHUNK, D), jnp.float32)),
        mesh=mesh,
        scratch_types=[pltpu.VMEM((N_CHUNK, IDX_MINOR), jnp.int32),
                       pltpu.VMEM((2, POS_CHUNK), jnp.int32),
                       pltpu.VMEM((HALF_ROWS, P), jnp.float32),
                       pltpu.VMEM((POS_CHUNK, D), jnp.float32),
                       pltpu.SemaphoreType.DMA,
                       pltpu.SemaphoreType.DMA],
        compiler_params=pltpu.CompilerParams(use_tc_tiling_on_sc=False),
    )
    def gather(img_hbm, rix_hbm, tab_hbm, pid_hbm, outx_hbm, outp_hbm,
               idx_v, pidx_v, xbuf_v, pbuf_v, sem, psem):
        b = lax.axis_index("s") * _NC + lax.axis_index("c")
        pltpu.sync_copy(rix_hbm.at[b], idx_v)
        pltpu.sync_copy(pid_hbm.at[b], pidx_v)
        GRPS = 3
        PER = 9

        def half(h):
            pd = pltpu.async_copy(tab_hbm.at[pidx_v.at[h]], pbuf_v, psem)

            def grp(g, carry):
                descs = []
                for t in range(PER):
                    j = g * PER + t
                    descs.append(pltpu.async_copy(
                        img_hbm.at[idx_v.at[h * HALF_CHUNK + j]],
                        xbuf_v.at[pl.ds(j * IDX_MINOR, IDX_MINOR)], sem))
                for d in descs:
                    d.wait()
                return carry

            lax.fori_loop(0, GRPS, grp, 0)
            pltpu.sync_copy(xbuf_v, outx_hbm.at[b, h])
            pd.wait()
            pltpu.sync_copy(pbuf_v, outp_hbm.at[b, h])

        half(0)
        half(1)

    return gather


_KSPLIT = IN_FEAT // 128


def _mm_body(x_ref, w_ref, bias_ref, o_ref):
    x6 = x_ref[...].astype(jnp.bfloat16).reshape(_MM_BM, _KSPLIT, 128)
    out = bias_ref[...].astype(jnp.float32)
    for k in range(_KSPLIT):
        out = out + jnp.dot(x6[:, k, :], w_ref[k],
                            preferred_element_type=jnp.float32)
    o_ref[...] = out


_MM_BM = 512
_mm = pl.pallas_call(
    _mm_body,
    grid=(B * LEN_KEEP // _MM_BM,),
    in_specs=[pl.BlockSpec((_MM_BM * _KSPLIT, 128), lambda i: (i, 0)),
              pl.BlockSpec((_KSPLIT, 128, D), lambda i: (0, 0, 0)),
              pl.BlockSpec((1, D), lambda i: (0, 0))],
    out_specs=pl.BlockSpec((_MM_BM, D), lambda i: (i, 0)),
    out_shape=jax.ShapeDtypeStruct((B * LEN_KEEP, D), jnp.float32),
)


def kernel(images, ldmks, body_bboxes, face_bboxes, noise, W_embed, b_embed,
           square_pos_embed, pos_cls_embed):
    del ldmks
    restore3, mask3, keep3, pos3, rix3, face3 = _a(
        noise.reshape(B, 1, NP_), body_bboxes.reshape(B, 1, 4),
        face_bboxes.reshape(B, 1, 4))
    ids_restore = restore3.reshape(B, NP_)
    mask = mask3.reshape(B, NP_)
    ids_keep = keep3.reshape(B, LEN_KEEP)
    face = face3.reshape(B, 4)
    pos_idx = pos3.reshape(B, 2, POS_CHUNK)
    rix = rix3.reshape(B, N_CHUNK, IDX_MINOR)
    img_rows = images.reshape(B * C * H * NS, P)
    xg, pos2 = _sc_gather()(img_rows, rix, square_pos_embed, pos_idx)
    pos_embed_masked = pos2.reshape(B, LEN_KEEP, D)
    Wp = W_embed.reshape(P, P, C, D).transpose(2, 0, 1, 3).reshape(IN_FEAT, D)
    W6 = Wp.astype(jnp.bfloat16).reshape(_KSPLIT, 128, D)
    x = _mm(xg.reshape(B * LEN_KEEP * _KSPLIT, 128), W6, b_embed.reshape(1, D))
    x = x.reshape(B, LEN_KEEP, D)
    full_imp = jnp.ones((B, NP_), jnp.float32)
    imp_masked = jnp.ones((B, LEN_KEEP), jnp.float32)
    return (x, mask, pos_embed_masked, pos_cls_embed, imp_masked,
            full_imp, ids_keep, ids_restore, square_pos_embed, face)


# speedup vs baseline: 1.0482x; 1.0482x over previous
import functools

import numpy as np
import jax
import jax.numpy as jnp
from jax import lax
from jax.experimental import pallas as pl
from jax.experimental.pallas import tpu as pltpu
from jax.experimental.pallas import tpu_sc as plsc

B = 32
C = 3
H = 384
WI = 384
P = 16
NS = 24
NP_ = NS * NS
D = 768
IN_FEAT = P * P * C
LEN_KEEP = 144
RPP = C * P
RPB = LEN_KEEP * RPP
IDX_MINOR = 128
N_CHUNK = RPB // IDX_MINOR
HALF_CHUNK = N_CHUNK // 2
HALF_ROWS = HALF_CHUNK * IDX_MINOR
POS_CHUNK = 72


def _patch_hw_rows():
    j = lax.broadcasted_iota(jnp.int32, (1, NP_), 1)
    return (j // NS).astype(jnp.float32), (j % NS).astype(jnp.float32)


def _snap_cols(b4):
    s = jnp.round(b4 * float(NS)) / float(NS)
    x0 = jnp.clip(s[:, 0:1], 0.0, 1.0 - 1.0 / NS)
    y0 = jnp.clip(s[:, 1:2], 0.0, 1.0 - 1.0 / NS)
    x1 = jnp.clip(jnp.maximum(s[:, 2:3], x0 + 1.0 / NS), 0.0, 1.0)
    y1 = jnp.clip(jnp.maximum(s[:, 3:4], y0 + 1.0 / NS), 0.0, 1.0)
    return x0, y0, x1, y1


def _a_body(n3_ref, bb3_ref, fb3_ref, restore_ref, mask_ref, keep_ref,
            pos_ref, rix_ref, face_ref):
    b = pl.program_id(0)
    bbr = bb3_ref[0]
    fbr = fb3_ref[0]
    x0, y0, x1, y1 = _snap_cols(bbr)
    bw = x1 - x0
    bh = y1 - y0
    f0 = (fbr[:, 0:1] - x0) / bw
    f1 = (fbr[:, 1:2] - y0) / bh
    f2 = (fbr[:, 2:3] - x0) / bw
    f3 = (fbr[:, 3:4] - y0) / bh
    fcl = jnp.clip(jnp.concatenate([f0, f1, f2, f3], axis=1), 0.0, 1.0)
    g0, g1, g2, g3 = _snap_cols(fcl)
    face_ref[...] = jnp.concatenate(
        [g0 * bw + x0, g1 * bh + y0, g2 * bw + x0, g3 * bh + y0], axis=1)[None]
    yb, xb = _patch_hw_rows()
    gx = (xb + 0.5) / NS
    gy = (yb + 0.5) / NS
    in_x = (gx >= x0) & (gx < x1)
    in_y = (gy >= y0) & (gy < y1)
    mx = jnp.clip(jnp.floor((gx - x0) / bw * NS), 0.0, float(NS - 1))
    my = jnp.clip(jnp.floor((gy - y0) / bh * NS), 0.0, float(NS - 1))
    ix = jnp.where(in_x, mx, xb)
    iy = jnp.where(in_y, my, yb)
    flat_row = iy * float(NS) + ix
    nrow = n3_ref[0]
    ncol = jnp.swapaxes(nrow, 0, 1)
    ilt = (lax.broadcasted_iota(jnp.int32, (NP_, NP_), 0)
           < lax.broadcasted_iota(jnp.int32, (NP_, NP_), 1))
    lt = ncol < nrow
    eq = ncol == nrow
    cmpf = (lt | (eq & ilt)).astype(jnp.float32)
    rank_row = jnp.sum(cmpf, axis=0, keepdims=True)
    restore_ref[...] = rank_row.astype(jnp.int32)[None]
    mask_ref[...] = (rank_row >= float(LEN_KEEP)).astype(jnp.float32)[None]
    r_iota = lax.broadcasted_iota(jnp.int32, (LEN_KEEP, NP_), 0
                                  ).astype(jnp.float32)
    m3 = (r_iota == rank_row).astype(jnp.float32)
    j_row = lax.broadcasted_iota(jnp.int32, (1, NP_), 1).astype(jnp.float32)
    h_row, w_row = _patch_hw_rows()
    keep_col = jnp.sum(m3 * j_row, axis=1, keepdims=True)
    pos_col = jnp.sum(m3 * flat_row, axis=1, keepdims=True)
    h_col = jnp.sum(m3 * h_row, axis=1, keepdims=True)
    w_col = jnp.sum(m3 * w_row, axis=1, keepdims=True)
    keep_ref[...] = keep_col.astype(jnp.int32)[None]
    pos_ref[...] = pos_col.astype(jnp.int32)[None]
    cp = lax.broadcasted_iota(jnp.int32, (1, RPP), 1)
    base48 = ((cp // P) * (H * NS) + (cp % P) * NS).astype(jnp.float32)
    rix = (h_col * float(P * NS) + w_col + base48
           ).astype(jnp.int32) + b * (C * H * NS)
    rix_ref[...] = rix[None]


_a = pl.pallas_call(
    _a_body,
    grid=(B,),
    in_specs=[pl.BlockSpec((1, 1, NP_), lambda b: (b, 0, 0)),
              pl.BlockSpec((1, 1, 4), lambda b: (b, 0, 0)),
              pl.BlockSpec((1, 1, 4), lambda b: (b, 0, 0))],
    out_specs=[pl.BlockSpec((1, 1, NP_), lambda b: (b, 0, 0)),
               pl.BlockSpec((1, 1, NP_), lambda b: (b, 0, 0)),
               pl.BlockSpec((1, LEN_KEEP, 1), lambda b: (b, 0, 0)),
               pl.BlockSpec((1, LEN_KEEP, 1), lambda b: (b, 0, 0)),
               pl.BlockSpec((1, LEN_KEEP, RPP), lambda b: (b, 0, 0)),
               pl.BlockSpec((1, 1, 4), lambda b: (b, 0, 0))],
    out_shape=(jax.ShapeDtypeStruct((B, 1, NP_), jnp.int32),
               jax.ShapeDtypeStruct((B, 1, NP_), jnp.float32),
               jax.ShapeDtypeStruct((B, LEN_KEEP, 1), jnp.int32),
               jax.ShapeDtypeStruct((B, LEN_KEEP, 1), jnp.int32),
               jax.ShapeDtypeStruct((B, LEN_KEEP, RPP), jnp.int32),
               jax.ShapeDtypeStruct((B, 1, 4), jnp.float32)),
)


_NC = 2


@functools.lru_cache(maxsize=None)
def _sc_gather():
    mesh = plsc.VectorSubcoreMesh(core_axis_name="c", subcore_axis_name="s")

    @functools.partial(
        pl.kernel,
        out_type=(jax.ShapeDtypeStruct((B, 2, HALF_ROWS, P), jnp.float32),
                  jax.ShapeDtypeStruct((B, 2, POS_CHUNK, D), jnp.float32)),
        mesh=mesh,
        scratch_types=[pltpu.VMEM((N_CHUNK, IDX_MINOR), jnp.int32),
                       pltpu.VMEM((2, POS_CHUNK), jnp.int32),
                       pltpu.VMEM((HALF_ROWS, P), jnp.float32),
                       pltpu.VMEM((POS_CHUNK, D), jnp.float32),
                       pltpu.SemaphoreType.DMA,
                       pltpu.SemaphoreType.DMA],
        compiler_params=pltpu.CompilerParams(use_tc_tiling_on_sc=False),
    )
    def gather(img_hbm, rix_hbm, tab_hbm, pid_hbm, outx_hbm, outp_hbm,
               idx_v, pidx_v, xbuf_v, pbuf_v, sem, psem):
        b = lax.axis_index("s") * _NC + lax.axis_index("c")
        pltpu.sync_copy(rix_hbm.at[b], idx_v)
        pltpu.sync_copy(pid_hbm.at[b], pidx_v)
        GRPS = 3
        PER = 9

        def half(h):
            pd = pltpu.async_copy(tab_hbm.at[pidx_v.at[h]], pbuf_v, psem)

            def grp(g, carry):
                descs = []
                for t in range(PER):
                    j = g * PER + t
                    descs.append(pltpu.async_copy(
                        img_hbm.at[idx_v.at[h * HALF_CHUNK + j]],
                        xbuf_v.at[pl.ds(j * IDX_MINOR, IDX_MINOR)], sem))
                for d in descs:
                    d.wait()
                return carry

            lax.fori_loop(0, GRPS, grp, 0)
            pltpu.sync_copy(xbuf_v, outx_hbm.at[b, h])
            pd.wait()
            pltpu.sync_copy(pbuf_v, outp_hbm.at[b, h])

        half(0)
        half(1)

    return gather


_KSPLIT = IN_FEAT // 128


def _mm_body(x_ref, w_ref, bias_ref, o_ref):
    x6 = x_ref[...].astype(jnp.bfloat16).reshape(_MM_BM, _KSPLIT, 128)
    out = bias_ref[...].astype(jnp.float32)
    for k in range(_KSPLIT):
        out = out + jnp.dot(x6[:, k, :], w_ref[k],
                            preferred_element_type=jnp.float32)
    o_ref[...] = out


_MM_BM = 512
_mm = pl.pallas_call(
    _mm_body,
    grid=(B * LEN_KEEP // _MM_BM,),
    in_specs=[pl.BlockSpec((_MM_BM * _KSPLIT, 128), lambda i: (i, 0)),
              pl.BlockSpec((_KSPLIT, 128, D), lambda i: (0, 0, 0)),
              pl.BlockSpec((1, D), lambda i: (0, 0))],
    out_specs=pl.BlockSpec((_MM_BM, D), lambda i: (i, 0)),
    out_shape=jax.ShapeDtypeStruct((B * LEN_KEEP, D), jnp.float32),
)


def kernel(images, ldmks, body_bboxes, face_bboxes, noise, W_embed, b_embed,
           square_pos_embed, pos_cls_embed):
    del ldmks
    restore3, mask3, keep3, pos3, rix3, face3 = _a(
        noise.reshape(B, 1, NP_), body_bboxes.reshape(B, 1, 4),
        face_bboxes.reshape(B, 1, 4))
    ids_restore = restore3.reshape(B, NP_)
    mask = mask3.reshape(B, NP_)
    ids_keep = keep3.reshape(B, LEN_KEEP)
    face = face3.reshape(B, 4)
    pos_idx = pos3.reshape(B, 2, POS_CHUNK)
    rix = rix3.reshape(B, N_CHUNK, IDX_MINOR)
    img_rows = images.reshape(B * C * H * NS, P)
    xg, pos2 = _sc_gather()(img_rows, rix, square_pos_embed, pos_idx)
    pos_embed_masked = pos2.reshape(B, LEN_KEEP, D)
    Wp = W_embed.reshape(P, P, C, D).transpose(2, 0, 1, 3).reshape(IN_FEAT, D)
    W6 = Wp.astype(jnp.bfloat16).reshape(_KSPLIT, 128, D)
    x = _mm(xg.reshape(B * LEN_KEEP * _KSPLIT, 128), W6, b_embed.reshape(1, D))
    x = x.reshape(B, LEN_KEEP, D)
    full_imp = jnp.ones((B, NP_), jnp.float32)
    imp_masked = jnp.ones((B, LEN_KEEP), jnp.float32)
    return (x, mask, pos_embed_masked, pos_cls_embed, imp_masked,
            full_imp, ids_keep, ids_restore, square_pos_embed, face)


# speedup vs baseline: 1.2307x; 1.1740x over previous
import functools

import numpy as np
import jax
import jax.numpy as jnp
from jax import lax
from jax.experimental import pallas as pl
from jax.experimental.pallas import tpu as pltpu
from jax.experimental.pallas import tpu_sc as plsc

B = 32
C = 3
H = 384
WI = 384
P = 16
NS = 24
NP_ = NS * NS
D = 768
IN_FEAT = P * P * C
LEN_KEEP = 144
RPP = C * P
RPB = LEN_KEEP * RPP
IDX_MINOR = 128
N_CHUNK = RPB // IDX_MINOR
HALF_CHUNK = N_CHUNK // 2
HALF_ROWS = HALF_CHUNK * IDX_MINOR
POS_CHUNK = 72


def _patch_hw_rows():
    j = lax.broadcasted_iota(jnp.int32, (1, NP_), 1)
    return (j // NS).astype(jnp.float32), (j % NS).astype(jnp.float32)


def _snap_cols(b4):
    s = jnp.round(b4 * float(NS)) / float(NS)
    x0 = jnp.clip(s[:, 0:1], 0.0, 1.0 - 1.0 / NS)
    y0 = jnp.clip(s[:, 1:2], 0.0, 1.0 - 1.0 / NS)
    x1 = jnp.clip(jnp.maximum(s[:, 2:3], x0 + 1.0 / NS), 0.0, 1.0)
    y1 = jnp.clip(jnp.maximum(s[:, 3:4], y0 + 1.0 / NS), 0.0, 1.0)
    return x0, y0, x1, y1


def _a_body(n3_ref, bb3_ref, fb3_ref, restore_ref, mask_ref, keep_ref,
            pos_ref, rix_ref, face_ref):
    b = pl.program_id(0)
    bbr = bb3_ref[0]
    fbr = fb3_ref[0]
    x0, y0, x1, y1 = _snap_cols(bbr)
    bw = x1 - x0
    bh = y1 - y0
    f0 = (fbr[:, 0:1] - x0) / bw
    f1 = (fbr[:, 1:2] - y0) / bh
    f2 = (fbr[:, 2:3] - x0) / bw
    f3 = (fbr[:, 3:4] - y0) / bh
    fcl = jnp.clip(jnp.concatenate([f0, f1, f2, f3], axis=1), 0.0, 1.0)
    g0, g1, g2, g3 = _snap_cols(fcl)
    face_ref[...] = jnp.concatenate(
        [g0 * bw + x0, g1 * bh + y0, g2 * bw + x0, g3 * bh + y0], axis=1)[None]
    yb, xb = _patch_hw_rows()
    gx = (xb + 0.5) / NS
    gy = (yb + 0.5) / NS
    in_x = (gx >= x0) & (gx < x1)
    in_y = (gy >= y0) & (gy < y1)
    mx = jnp.clip(jnp.floor((gx - x0) / bw * NS), 0.0, float(NS - 1))
    my = jnp.clip(jnp.floor((gy - y0) / bh * NS), 0.0, float(NS - 1))
    ix = jnp.where(in_x, mx, xb)
    iy = jnp.where(in_y, my, yb)
    flat_row = iy * float(NS) + ix
    nrow = n3_ref[0]
    ncol = jnp.swapaxes(nrow, 0, 1)
    ilt = (lax.broadcasted_iota(jnp.int32, (NP_, NP_), 0)
           < lax.broadcasted_iota(jnp.int32, (NP_, NP_), 1))
    lt = ncol < nrow
    eq = ncol == nrow
    cmpf = (lt | (eq & ilt)).astype(jnp.float32)
    rank_row = jnp.sum(cmpf, axis=0, keepdims=True)
    restore_ref[...] = rank_row.astype(jnp.int32)[None]
    mask_ref[...] = (rank_row >= float(LEN_KEEP)).astype(jnp.float32)[None]
    r_iota = lax.broadcasted_iota(jnp.int32, (LEN_KEEP, NP_), 0
                                  ).astype(jnp.float32)
    m3 = (r_iota == rank_row).astype(jnp.float32)
    j_row = lax.broadcasted_iota(jnp.int32, (1, NP_), 1).astype(jnp.float32)
    h_row, w_row = _patch_hw_rows()
    keep_col = jnp.sum(m3 * j_row, axis=1, keepdims=True)
    pos_col = jnp.sum(m3 * flat_row, axis=1, keepdims=True)
    h_col = jnp.sum(m3 * h_row, axis=1, keepdims=True)
    w_col = jnp.sum(m3 * w_row, axis=1, keepdims=True)
    keep_ref[...] = keep_col.astype(jnp.int32)[None]
    pos_ref[...] = pos_col.astype(jnp.int32)[None]
    cp = lax.broadcasted_iota(jnp.int32, (1, RPP), 1)
    base48 = ((cp // P) * (H * NS) + (cp % P) * NS).astype(jnp.float32)
    rix = (h_col * float(P * NS) + w_col + base48
           ).astype(jnp.int32) + b * (C * H * NS)
    rix_ref[...] = rix[None]


_a = pl.pallas_call(
    _a_body,
    grid=(B,),
    in_specs=[pl.BlockSpec((1, 1, NP_), lambda b: (b, 0, 0)),
              pl.BlockSpec((1, 1, 4), lambda b: (b, 0, 0)),
              pl.BlockSpec((1, 1, 4), lambda b: (b, 0, 0))],
    out_specs=[pl.BlockSpec((1, 1, NP_), lambda b: (b, 0, 0)),
               pl.BlockSpec((1, 1, NP_), lambda b: (b, 0, 0)),
               pl.BlockSpec((1, LEN_KEEP, 1), lambda b: (b, 0, 0)),
               pl.BlockSpec((1, LEN_KEEP, 1), lambda b: (b, 0, 0)),
               pl.BlockSpec((1, LEN_KEEP, RPP), lambda b: (b, 0, 0)),
               pl.BlockSpec((1, 1, 4), lambda b: (b, 0, 0))],
    out_shape=(jax.ShapeDtypeStruct((B, 1, NP_), jnp.int32),
               jax.ShapeDtypeStruct((B, 1, NP_), jnp.float32),
               jax.ShapeDtypeStruct((B, LEN_KEEP, 1), jnp.int32),
               jax.ShapeDtypeStruct((B, LEN_KEEP, 1), jnp.int32),
               jax.ShapeDtypeStruct((B, LEN_KEEP, RPP), jnp.int32),
               jax.ShapeDtypeStruct((B, 1, 4), jnp.float32)),
)


_NC = 2


@functools.lru_cache(maxsize=None)
def _sc_kernels():
    mesh = plsc.VectorSubcoreMesh(core_axis_name="c", subcore_axis_name="s")

    @functools.partial(
        pl.kernel,
        out_type=jax.ShapeDtypeStruct((B, RPB, P), jnp.float32),
        mesh=mesh,
        scratch_types=[pltpu.VMEM((N_CHUNK, IDX_MINOR), jnp.int32),
                       pltpu.VMEM((RPB, P), jnp.float32),
                       pltpu.SemaphoreType.DMA],
        compiler_params=pltpu.CompilerParams(use_tc_tiling_on_sc=False),
    )
    def patch_gather(img_hbm, rix_hbm, outx_hbm, idx_v, xbuf_v, sem):
        b = lax.axis_index("s") * _NC + lax.axis_index("c")
        pltpu.sync_copy(rix_hbm.at[b], idx_v)
        GRPS = 6
        PER = 9

        def grp(g, carry):
            descs = []
            for t in range(PER):
                j = g * PER + t
                descs.append(pltpu.async_copy(
                    img_hbm.at[idx_v.at[j]],
                    xbuf_v.at[pl.ds(j * IDX_MINOR, IDX_MINOR)], sem))
            for d in descs:
                d.wait()
            return carry

        lax.fori_loop(0, GRPS, grp, 0)
        pltpu.sync_copy(xbuf_v, outx_hbm.at[b])

    @functools.partial(
        pl.kernel,
        out_type=jax.ShapeDtypeStruct((B, LEN_KEEP, D), jnp.float32),
        mesh=mesh,
        scratch_types=[pltpu.VMEM((2, POS_CHUNK), jnp.int32),
                       pltpu.VMEM((LEN_KEEP, D), jnp.float32),
                       pltpu.SemaphoreType.DMA],
    )
    def pos_gather(tab_hbm, pid_hbm, outp_hbm, pidx_v, pbuf_v, sem):
        b = lax.axis_index("s") * _NC + lax.axis_index("c")
        pltpu.sync_copy(pid_hbm.at[b], pidx_v)
        d0 = pltpu.async_copy(tab_hbm.at[pidx_v.at[0]],
                              pbuf_v.at[pl.ds(0, POS_CHUNK)], sem)
        d1 = pltpu.async_copy(tab_hbm.at[pidx_v.at[1]],
                              pbuf_v.at[pl.ds(POS_CHUNK, POS_CHUNK)], sem)
        d0.wait()
        d1.wait()
        pltpu.sync_copy(pbuf_v, outp_hbm.at[b])

    return patch_gather, pos_gather


_KSPLIT = IN_FEAT // 128


def _mm_body(x_ref, w_ref, bias_ref, o_ref):
    x6 = x_ref[...].astype(jnp.bfloat16).reshape(_MM_BM, _KSPLIT, 128)
    out = bias_ref[...].astype(jnp.float32)
    for k in range(_KSPLIT):
        out = out + jnp.dot(x6[:, k, :], w_ref[k],
                            preferred_element_type=jnp.float32)
    o_ref[...] = out


_MM_BM = 512
_mm = pl.pallas_call(
    _mm_body,
    grid=(B * LEN_KEEP // _MM_BM,),
    in_specs=[pl.BlockSpec((_MM_BM * _KSPLIT, 128), lambda i: (i, 0)),
              pl.BlockSpec((_KSPLIT, 128, D), lambda i: (0, 0, 0)),
              pl.BlockSpec((1, D), lambda i: (0, 0))],
    out_specs=pl.BlockSpec((_MM_BM, D), lambda i: (i, 0)),
    out_shape=jax.ShapeDtypeStruct((B * LEN_KEEP, D), jnp.float32),
)


def kernel(images, ldmks, body_bboxes, face_bboxes, noise, W_embed, b_embed,
           square_pos_embed, pos_cls_embed):
    del ldmks
    restore3, mask3, keep3, pos3, rix3, face3 = _a(
        noise.reshape(B, 1, NP_), body_bboxes.reshape(B, 1, 4),
        face_bboxes.reshape(B, 1, 4))
    ids_restore = restore3.reshape(B, NP_)
    mask = mask3.reshape(B, NP_)
    ids_keep = keep3.reshape(B, LEN_KEEP)
    face = face3.reshape(B, 4)
    pos_idx = pos3.reshape(B, 2, POS_CHUNK)
    rix = rix3.reshape(B, N_CHUNK, IDX_MINOR)
    img_rows = images.reshape(B * C * H * NS, P)
    patch_gather, pos_gather = _sc_kernels()
    xg = patch_gather(img_rows, rix)
    pos_embed_masked = pos_gather(square_pos_embed, pos_idx)
    Wp = W_embed.reshape(P, P, C, D).transpose(2, 0, 1, 3).reshape(IN_FEAT, D)
    W6 = Wp.astype(jnp.bfloat16).reshape(_KSPLIT, 128, D)
    x = _mm(xg.reshape(B * LEN_KEEP * _KSPLIT, 128), W6, b_embed.reshape(1, D))
    x = x.reshape(B, LEN_KEEP, D)
    full_imp = jnp.ones((B, NP_), jnp.float32)
    imp_masked = jnp.ones((B, LEN_KEEP), jnp.float32)
    return (x, mask, pos_embed_masked, pos_cls_embed, imp_masked,
            full_imp, ids_keep, ids_restore, square_pos_embed, face)


# speedup vs baseline: 1.4723x; 1.1964x over previous
import functools

import numpy as np
import jax
import jax.numpy as jnp
from jax import lax
from jax.experimental import pallas as pl
from jax.experimental.pallas import tpu as pltpu
from jax.experimental.pallas import tpu_sc as plsc

B = 32
C = 3
H = 384
WI = 384
P = 16
NS = 24
NP_ = NS * NS
D = 768
IN_FEAT = P * P * C
LEN_KEEP = 144
RPP = C * P
RPB = LEN_KEEP * RPP
IDX_MINOR = 128
N_CHUNK = RPB // IDX_MINOR
HALF_CHUNK = N_CHUNK // 2
HALF_ROWS = HALF_CHUNK * IDX_MINOR
POS_CHUNK = 72


def _patch_hw_rows():
    j = lax.broadcasted_iota(jnp.int32, (1, NP_), 1)
    return (j // NS).astype(jnp.float32), (j % NS).astype(jnp.float32)


def _snap_cols(b4):
    s = jnp.round(b4 * float(NS)) / float(NS)
    x0 = jnp.clip(s[:, 0:1], 0.0, 1.0 - 1.0 / NS)
    y0 = jnp.clip(s[:, 1:2], 0.0, 1.0 - 1.0 / NS)
    x1 = jnp.clip(jnp.maximum(s[:, 2:3], x0 + 1.0 / NS), 0.0, 1.0)
    y1 = jnp.clip(jnp.maximum(s[:, 3:4], y0 + 1.0 / NS), 0.0, 1.0)
    return x0, y0, x1, y1


def _a_body(n3_ref, bb3_ref, fb3_ref, img_ref, restore_ref, mask_ref,
            keep_ref, pos_ref, rix_ref, face_ref, lin_ref):
    b = pl.program_id(0)
    lin_ref[...] = img_ref[...].reshape(C * H * WI // 128, 128)
    bbr = bb3_ref[0]
    fbr = fb3_ref[0]
    x0, y0, x1, y1 = _snap_cols(bbr)
    bw = x1 - x0
    bh = y1 - y0
    f0 = (fbr[:, 0:1] - x0) / bw
    f1 = (fbr[:, 1:2] - y0) / bh
    f2 = (fbr[:, 2:3] - x0) / bw
    f3 = (fbr[:, 3:4] - y0) / bh
    fcl = jnp.clip(jnp.concatenate([f0, f1, f2, f3], axis=1), 0.0, 1.0)
    g0, g1, g2, g3 = _snap_cols(fcl)
    face_ref[...] = jnp.concatenate(
        [g0 * bw + x0, g1 * bh + y0, g2 * bw + x0, g3 * bh + y0], axis=1)[None]
    yb, xb = _patch_hw_rows()
    gx = (xb + 0.5) / NS
    gy = (yb + 0.5) / NS
    in_x = (gx >= x0) & (gx < x1)
    in_y = (gy >= y0) & (gy < y1)
    mx = jnp.clip(jnp.floor((gx - x0) / bw * NS), 0.0, float(NS - 1))
    my = jnp.clip(jnp.floor((gy - y0) / bh * NS), 0.0, float(NS - 1))
    ix = jnp.where(in_x, mx, xb)
    iy = jnp.where(in_y, my, yb)
    flat_row = iy * float(NS) + ix
    nrow = n3_ref[0]
    ncol = jnp.swapaxes(nrow, 0, 1)
    ilt = (lax.broadcasted_iota(jnp.int32, (NP_, NP_), 0)
           < lax.broadcasted_iota(jnp.int32, (NP_, NP_), 1))
    lt = ncol < nrow
    eq = ncol == nrow
    cmpf = (lt | (eq & ilt)).astype(jnp.float32)
    rank_row = jnp.sum(cmpf, axis=0, keepdims=True)
    restore_ref[...] = rank_row.astype(jnp.int32)[None]
    mask_ref[...] = (rank_row >= float(LEN_KEEP)).astype(jnp.float32)[None]
    r_iota = lax.broadcasted_iota(jnp.int32, (LEN_KEEP, NP_), 0
                                  ).astype(jnp.float32)
    m3 = (r_iota == rank_row).astype(jnp.float32)
    j_row = lax.broadcasted_iota(jnp.int32, (1, NP_), 1).astype(jnp.float32)
    h_row, w_row = _patch_hw_rows()
    keep_col = jnp.sum(m3 * j_row, axis=1, keepdims=True)
    pos_col = jnp.sum(m3 * flat_row, axis=1, keepdims=True)
    h_col = jnp.sum(m3 * h_row, axis=1, keepdims=True)
    w_col = jnp.sum(m3 * w_row, axis=1, keepdims=True)
    keep_ref[...] = keep_col.astype(jnp.int32)[None]
    pos_ref[...] = pos_col.astype(jnp.int32)[None]
    cp = lax.broadcasted_iota(jnp.int32, (1, RPP), 1)
    base48 = ((cp // P) * (H * NS) + (cp % P) * NS).astype(jnp.float32)
    rix = (h_col * float(P * NS) + w_col + base48
           ).astype(jnp.int32) + b * (C * H * NS)
    rix_ref[...] = rix[None]


_a = pl.pallas_call(
    _a_body,
    grid=(B,),
    in_specs=[pl.BlockSpec((1, 1, NP_), lambda b: (b, 0, 0)),
              pl.BlockSpec((1, 1, 4), lambda b: (b, 0, 0)),
              pl.BlockSpec((1, 1, 4), lambda b: (b, 0, 0)),
              pl.BlockSpec((1, C, H, WI), lambda b: (b, 0, 0, 0))],
    out_specs=[pl.BlockSpec((1, 1, NP_), lambda b: (b, 0, 0)),
               pl.BlockSpec((1, 1, NP_), lambda b: (b, 0, 0)),
               pl.BlockSpec((1, LEN_KEEP, 1), lambda b: (b, 0, 0)),
               pl.BlockSpec((1, LEN_KEEP, 1), lambda b: (b, 0, 0)),
               pl.BlockSpec((1, LEN_KEEP, RPP), lambda b: (b, 0, 0)),
               pl.BlockSpec((1, 1, 4), lambda b: (b, 0, 0)),
               pl.BlockSpec((C * H * WI // 128, 128), lambda b: (b, 0))],
    out_shape=(jax.ShapeDtypeStruct((B, 1, NP_), jnp.int32),
               jax.ShapeDtypeStruct((B, 1, NP_), jnp.float32),
               jax.ShapeDtypeStruct((B, LEN_KEEP, 1), jnp.int32),
               jax.ShapeDtypeStruct((B, LEN_KEEP, 1), jnp.int32),
               jax.ShapeDtypeStruct((B, LEN_KEEP, RPP), jnp.int32),
               jax.ShapeDtypeStruct((B, 1, 4), jnp.float32),
               jax.ShapeDtypeStruct((B * C * H * WI // 128, 128),
                                    jnp.float32)),
)


_NC = 2


@functools.lru_cache(maxsize=None)
def _sc_kernels():
    mesh = plsc.VectorSubcoreMesh(core_axis_name="c", subcore_axis_name="s")

    @functools.partial(
        pl.kernel,
        out_type=jax.ShapeDtypeStruct((B, RPB, P), jnp.float32),
        mesh=mesh,
        scratch_types=[pltpu.VMEM((N_CHUNK, IDX_MINOR), jnp.int32),
                       pltpu.VMEM((RPB, P), jnp.float32),
                       pltpu.SemaphoreType.DMA],
        compiler_params=pltpu.CompilerParams(use_tc_tiling_on_sc=False),
    )
    def patch_gather(img_hbm, rix_hbm, outx_hbm, idx_v, xbuf_v, sem):
        b = lax.axis_index("s") * _NC + lax.axis_index("c")
        pltpu.sync_copy(rix_hbm.at[b], idx_v)
        GRPS = 6
        PER = 9

        def grp(g, carry):
            descs = []
            for t in range(PER):
                j = g * PER + t
                descs.append(pltpu.async_copy(
                    img_hbm.at[idx_v.at[j]],
                    xbuf_v.at[pl.ds(j * IDX_MINOR, IDX_MINOR)], sem))
            for d in descs:
                d.wait()
            return carry

        lax.fori_loop(0, GRPS, grp, 0)
        pltpu.sync_copy(xbuf_v, outx_hbm.at[b])

    @functools.partial(
        pl.kernel,
        out_type=jax.ShapeDtypeStruct((B, LEN_KEEP, D), jnp.float32),
        mesh=mesh,
        scratch_types=[pltpu.VMEM((2, POS_CHUNK), jnp.int32),
                       pltpu.VMEM((LEN_KEEP, D), jnp.float32),
                       pltpu.SemaphoreType.DMA],
    )
    def pos_gather(tab_hbm, pid_hbm, outp_hbm, pidx_v, pbuf_v, sem):
        b = lax.axis_index("s") * _NC + lax.axis_index("c")
        pltpu.sync_copy(pid_hbm.at[b], pidx_v)
        d0 = pltpu.async_copy(tab_hbm.at[pidx_v.at[0]],
                              pbuf_v.at[pl.ds(0, POS_CHUNK)], sem)
        d1 = pltpu.async_copy(tab_hbm.at[pidx_v.at[1]],
                              pbuf_v.at[pl.ds(POS_CHUNK, POS_CHUNK)], sem)
        d0.wait()
        d1.wait()
        pltpu.sync_copy(pbuf_v, outp_hbm.at[b])

    return patch_gather, pos_gather


_KSPLIT = IN_FEAT // 128


def _mm_body(x_ref, w_ref, bias_ref, o_ref):
    x2 = x_ref[...].astype(jnp.bfloat16).reshape(_MM_BM, IN_FEAT)
    o_ref[...] = jnp.dot(x2, w_ref[...].reshape(IN_FEAT, D),
                         preferred_element_type=jnp.float32) + bias_ref[...]


_MM_BM = 512
_mm = pl.pallas_call(
    _mm_body,
    grid=(B * LEN_KEEP // _MM_BM,),
    in_specs=[pl.BlockSpec((_MM_BM * _KSPLIT, 128), lambda i: (i, 0)),
              pl.BlockSpec((_KSPLIT, 128, D), lambda i: (0, 0, 0)),
              pl.BlockSpec((1, D), lambda i: (0, 0))],
    out_specs=pl.BlockSpec((_MM_BM, D), lambda i: (i, 0)),
    out_shape=jax.ShapeDtypeStruct((B * LEN_KEEP, D), jnp.float32),
)


def kernel(images, ldmks, body_bboxes, face_bboxes, noise, W_embed, b_embed,
           square_pos_embed, pos_cls_embed):
    del ldmks
    restore3, mask3, keep3, pos3, rix3, face3, img_lin = _a(
        noise.reshape(B, 1, NP_), body_bboxes.reshape(B, 1, 4),
        face_bboxes.reshape(B, 1, 4), images)
    ids_restore = restore3.reshape(B, NP_)
    mask = mask3.reshape(B, NP_)
    ids_keep = keep3.reshape(B, LEN_KEEP)
    face = face3.reshape(B, 4)
    pos_idx = pos3.reshape(B, 2, POS_CHUNK)
    rix = rix3.reshape(B, N_CHUNK, IDX_MINOR)
    img_rows = img_lin.reshape(B * C * H * NS, P)
    patch_gather, pos_gather = _sc_kernels()
    xg = patch_gather(img_rows, rix)
    pos_embed_masked = pos_gather(square_pos_embed, pos_idx)
    Wp = W_embed.reshape(P, P, C, D).transpose(2, 0, 1, 3).reshape(IN_FEAT, D)
    W6 = Wp.astype(jnp.bfloat16).reshape(_KSPLIT, 128, D)
    x = _mm(xg.reshape(B * LEN_KEEP * _KSPLIT, 128), W6, b_embed.reshape(1, D))
    x = x.reshape(B, LEN_KEEP, D)
    full_imp = jnp.ones((B, NP_), jnp.float32)
    imp_masked = jnp.ones((B, LEN_KEEP), jnp.float32)
    return (x, mask, pos_embed_masked, pos_cls_embed, imp_masked,
            full_imp, ids_keep, ids_restore, square_pos_embed, face)


# speedup vs baseline: 1.5083x; 1.0244x over previous
import functools

import numpy as np
import jax
import jax.numpy as jnp
from jax import lax
from jax.experimental import pallas as pl
from jax.experimental.pallas import tpu as pltpu
from jax.experimental.pallas import tpu_sc as plsc

B = 32
C = 3
H = 384
WI = 384
P = 16
NS = 24
NP_ = NS * NS
D = 768
IN_FEAT = P * P * C
LEN_KEEP = 144
RPP = C * P
RPB = LEN_KEEP * RPP
IDX_MINOR = 128
N_CHUNK = RPB // IDX_MINOR
HALF_CHUNK = N_CHUNK // 2
HALF_ROWS = HALF_CHUNK * IDX_MINOR
POS_CHUNK = 72


def _patch_hw_rows():
    j = lax.broadcasted_iota(jnp.int32, (1, NP_), 1)
    return (j // NS).astype(jnp.float32), (j % NS).astype(jnp.float32)


def _snap_cols(b4):
    s = jnp.round(b4 * float(NS)) / float(NS)
    x0 = jnp.clip(s[:, 0:1], 0.0, 1.0 - 1.0 / NS)
    y0 = jnp.clip(s[:, 1:2], 0.0, 1.0 - 1.0 / NS)
    x1 = jnp.clip(jnp.maximum(s[:, 2:3], x0 + 1.0 / NS), 0.0, 1.0)
    y1 = jnp.clip(jnp.maximum(s[:, 3:4], y0 + 1.0 / NS), 0.0, 1.0)
    return x0, y0, x1, y1


def _a_body(n3_ref, bb3_ref, fb3_ref, img_ref, restore_ref, mask_ref,
            keep_ref, pos_ref, rix_ref, face_ref, lin_ref):
    b = pl.program_id(0)
    lin_ref[...] = img_ref[...].reshape(C * H * WI // 128, 128)
    bbr = bb3_ref[0]
    fbr = fb3_ref[0]
    x0, y0, x1, y1 = _snap_cols(bbr)
    bw = x1 - x0
    bh = y1 - y0
    f0 = (fbr[:, 0:1] - x0) / bw
    f1 = (fbr[:, 1:2] - y0) / bh
    f2 = (fbr[:, 2:3] - x0) / bw
    f3 = (fbr[:, 3:4] - y0) / bh
    fcl = jnp.clip(jnp.concatenate([f0, f1, f2, f3], axis=1), 0.0, 1.0)
    g0, g1, g2, g3 = _snap_cols(fcl)
    face_ref[...] = jnp.concatenate(
        [g0 * bw + x0, g1 * bh + y0, g2 * bw + x0, g3 * bh + y0], axis=1)[None]
    yb, xb = _patch_hw_rows()
    gx = (xb + 0.5) / NS
    gy = (yb + 0.5) / NS
    in_x = (gx >= x0) & (gx < x1)
    in_y = (gy >= y0) & (gy < y1)
    mx = jnp.clip(jnp.floor((gx - x0) / bw * NS), 0.0, float(NS - 1))
    my = jnp.clip(jnp.floor((gy - y0) / bh * NS), 0.0, float(NS - 1))
    ix = jnp.where(in_x, mx, xb)
    iy = jnp.where(in_y, my, yb)
    flat_row = iy * float(NS) + ix
    nrow = n3_ref[0]
    ncol = jnp.swapaxes(nrow, 0, 1)
    ilt = (lax.broadcasted_iota(jnp.int32, (NP_, NP_), 0)
           < lax.broadcasted_iota(jnp.int32, (NP_, NP_), 1))
    lt = ncol < nrow
    eq = ncol == nrow
    cmpf = (lt | (eq & ilt)).astype(jnp.float32)
    rank_row = jnp.sum(cmpf, axis=0, keepdims=True)
    restore_ref[...] = rank_row.astype(jnp.int32)[None]
    mask_ref[...] = (rank_row >= float(LEN_KEEP)).astype(jnp.float32)[None]
    r_iota = lax.broadcasted_iota(jnp.int32, (LEN_KEEP, NP_), 0
                                  ).astype(jnp.float32)
    m3 = (r_iota == rank_row).astype(jnp.float32)
    j_row = lax.broadcasted_iota(jnp.int32, (1, NP_), 1).astype(jnp.float32)
    h_row, w_row = _patch_hw_rows()
    keep_col = jnp.sum(m3 * j_row, axis=1, keepdims=True)
    pos_col = jnp.sum(m3 * flat_row, axis=1, keepdims=True)
    h_col = jnp.sum(m3 * h_row, axis=1, keepdims=True)
    w_col = jnp.sum(m3 * w_row, axis=1, keepdims=True)
    keep_ref[...] = keep_col.astype(jnp.int32)[None]
    pos_ref[...] = pos_col.astype(jnp.int32)[None]
    cp = lax.broadcasted_iota(jnp.int32, (1, RPP), 1)
    base48 = ((cp // P) * (H * NS) + (cp % P) * NS).astype(jnp.float32)
    rix = (h_col * float(P * NS) + w_col + base48
           ).astype(jnp.int32) + b * (C * H * NS)
    rix_ref[...] = rix[None]


_a = pl.pallas_call(
    _a_body,
    grid=(B,),
    in_specs=[pl.BlockSpec((1, 1, NP_), lambda b: (b, 0, 0)),
              pl.BlockSpec((1, 1, 4), lambda b: (b, 0, 0)),
              pl.BlockSpec((1, 1, 4), lambda b: (b, 0, 0)),
              pl.BlockSpec((1, C, H, WI), lambda b: (b, 0, 0, 0))],
    out_specs=[pl.BlockSpec((1, 1, NP_), lambda b: (b, 0, 0)),
               pl.BlockSpec((1, 1, NP_), lambda b: (b, 0, 0)),
               pl.BlockSpec((1, LEN_KEEP, 1), lambda b: (b, 0, 0)),
               pl.BlockSpec((1, LEN_KEEP, 1), lambda b: (b, 0, 0)),
               pl.BlockSpec((1, LEN_KEEP, RPP), lambda b: (b, 0, 0)),
               pl.BlockSpec((1, 1, 4), lambda b: (b, 0, 0)),
               pl.BlockSpec((C * H * WI // 128, 128), lambda b: (b, 0))],
    out_shape=(jax.ShapeDtypeStruct((B, 1, NP_), jnp.int32),
               jax.ShapeDtypeStruct((B, 1, NP_), jnp.float32),
               jax.ShapeDtypeStruct((B, LEN_KEEP, 1), jnp.int32),
               jax.ShapeDtypeStruct((B, LEN_KEEP, 1), jnp.int32),
               jax.ShapeDtypeStruct((B, LEN_KEEP, RPP), jnp.int32),
               jax.ShapeDtypeStruct((B, 1, 4), jnp.float32),
               jax.ShapeDtypeStruct((B * C * H * WI // 128, 128),
                                    jnp.float32)),
)


_NC = 2


@functools.lru_cache(maxsize=None)
def _sc_kernels():
    mesh = plsc.VectorSubcoreMesh(core_axis_name="c", subcore_axis_name="s")

    @functools.partial(
        pl.kernel,
        out_type=jax.ShapeDtypeStruct((B, RPB, P), jnp.float32),
        mesh=mesh,
        scratch_types=[pltpu.VMEM((N_CHUNK, IDX_MINOR), jnp.int32),
                       pltpu.VMEM((RPB, P), jnp.float32),
                       pltpu.SemaphoreType.DMA],
        compiler_params=pltpu.CompilerParams(use_tc_tiling_on_sc=False),
    )
    def patch_gather(img_hbm, rix_hbm, outx_hbm, idx_v, xbuf_v, sem):
        b = lax.axis_index("s") * _NC + lax.axis_index("c")
        pltpu.sync_copy(rix_hbm.at[b], idx_v)
        GRPS = 2
        PER = 27

        def grp(g, carry):
            descs = []
            for t in range(PER):
                j = g * PER + t
                descs.append(pltpu.async_copy(
                    img_hbm.at[idx_v.at[j]],
                    xbuf_v.at[pl.ds(j * IDX_MINOR, IDX_MINOR)], sem))
            for d in descs:
                d.wait()
            return carry

        lax.fori_loop(0, GRPS, grp, 0)
        pltpu.sync_copy(xbuf_v, outx_hbm.at[b])

    @functools.partial(
        pl.kernel,
        out_type=jax.ShapeDtypeStruct((B, LEN_KEEP, D), jnp.float32),
        mesh=mesh,
        scratch_types=[pltpu.VMEM((2, POS_CHUNK), jnp.int32),
                       pltpu.VMEM((LEN_KEEP, D), jnp.float32),
                       pltpu.SemaphoreType.DMA],
    )
    def pos_gather(tab_hbm, pid_hbm, outp_hbm, pidx_v, pbuf_v, sem):
        b = lax.axis_index("s") * _NC + lax.axis_index("c")
        pltpu.sync_copy(pid_hbm.at[b], pidx_v)
        d0 = pltpu.async_copy(tab_hbm.at[pidx_v.at[0]],
                              pbuf_v.at[pl.ds(0, POS_CHUNK)], sem)
        d1 = pltpu.async_copy(tab_hbm.at[pidx_v.at[1]],
                              pbuf_v.at[pl.ds(POS_CHUNK, POS_CHUNK)], sem)
        d0.wait()
        d1.wait()
        pltpu.sync_copy(pbuf_v, outp_hbm.at[b])

    return patch_gather, pos_gather


_KSPLIT = IN_FEAT // 128


def _mm_body(x_ref, w_ref, bias_ref, o_ref):
    x2 = x_ref[...].astype(jnp.bfloat16).reshape(_MM_BM, IN_FEAT)
    o_ref[...] = jnp.dot(x2, w_ref[...].reshape(IN_FEAT, D),
                         preferred_element_type=jnp.float32) + bias_ref[...]


_MM_BM = 1536
_mm = pl.pallas_call(
    _mm_body,
    grid=(B * LEN_KEEP // _MM_BM,),
    in_specs=[pl.BlockSpec((_MM_BM * _KSPLIT, 128), lambda i: (i, 0)),
              pl.BlockSpec((_KSPLIT, 128, D), lambda i: (0, 0, 0)),
              pl.BlockSpec((1, D), lambda i: (0, 0))],
    out_specs=pl.BlockSpec((_MM_BM, D), lambda i: (i, 0)),
    out_shape=jax.ShapeDtypeStruct((B * LEN_KEEP, D), jnp.float32),
)


def kernel(images, ldmks, body_bboxes, face_bboxes, noise, W_embed, b_embed,
           square_pos_embed, pos_cls_embed):
    del ldmks
    restore3, mask3, keep3, pos3, rix3, face3, img_lin = _a(
        noise.reshape(B, 1, NP_), body_bboxes.reshape(B, 1, 4),
        face_bboxes.reshape(B, 1, 4), images)
    ids_restore = restore3.reshape(B, NP_)
    mask = mask3.reshape(B, NP_)
    ids_keep = keep3.reshape(B, LEN_KEEP)
    face = face3.reshape(B, 4)
    pos_idx = pos3.reshape(B, 2, POS_CHUNK)
    rix = rix3.reshape(B, N_CHUNK, IDX_MINOR)
    img_rows = img_lin.reshape(B * C * H * NS, P)
    patch_gather, pos_gather = _sc_kernels()
    xg = patch_gather(img_rows, rix)
    pos_embed_masked = pos_gather(square_pos_embed, pos_idx)
    Wp = W_embed.reshape(P, P, C, D).transpose(2, 0, 1, 3).reshape(IN_FEAT, D)
    W6 = Wp.astype(jnp.bfloat16).reshape(_KSPLIT, 128, D)
    x = _mm(xg.reshape(B * LEN_KEEP * _KSPLIT, 128), W6, b_embed.reshape(1, D))
    x = x.reshape(B, LEN_KEEP, D)
    full_imp = jnp.ones((B, NP_), jnp.float32)
    imp_masked = jnp.ones((B, LEN_KEEP), jnp.float32)
    return (x, mask, pos_embed_masked, pos_cls_embed, imp_masked,
            full_imp, ids_keep, ids_restore, square_pos_embed, face)
